# baseline (speedup 1.0000x reference)
import functools

import jax
import jax.numpy as jnp
from jax import lax
from jax.experimental import pallas as pl
from jax.experimental.pallas import tpu as pltpu
from jax.experimental.pallas import tpu_sc as plsc

N = 10000
E = 320000
CN = 50
NG = 10
SP = 512

_NC, _NS = 2, 16
_NW = _NC * _NS
_EROWS = 80
_EPT = _EROWS * 128
_EPAD = _NW * _EPT
_ACC_ROWS = 10240
_RPT = _ACC_ROWS // _NS

_sc_mesh = plsc.VectorSubcoreMesh(core_axis_name="c", subcore_axis_name="s",
                                  num_cores=_NC, num_subcores=_NS)


def _zero_rows(buf, rows, cols):
    def zrow(i, _):
        for l in range(cols // 16):
            buf[i, pl.ds(l * 16, 16)] = jnp.zeros((16,), jnp.float32)
        return 0
    lax.fori_loop(0, rows, zrow, 0)


_ER = _EPAD // _NS // 128
_WIN = 2560
_ZROW = N


def _edge_body(g_hbm, src_hbm, dst_hbm, out_hbm,
               src_v, dst_v, rows_a, rows_b, zbuf, acc, sem_a, sem_b):
    c = lax.axis_index("c")
    s = lax.axis_index("s")

    _zero_rows(zbuf, 128, 128)

    def do_phase(base):
        pltpu.sync_copy(zbuf, acc.at[pl.ds(s * 160, 128)])
        pltpu.sync_copy(zbuf.at[pl.ds(0, 32)],
                        acc.at[pl.ds(s * 160 + 128, 32)])
        plsc.subcore_barrier()

        for b in range(2):
            pltpu.sync_copy(src_hbm.at[pl.ds(s * _ER + b * 80, 80)], src_v)
            pltpu.sync_copy(dst_hbm.at[pl.ds(s * _ER + b * 80, 80)], dst_v)

            def remap(j, _):
                for l in range(8):
                    sl = pl.ds(l * 16, 16)
                    dv = dst_v[j, sl]
                    sv = src_v[j, sl]
                    loc = dv - base
                    valid = (loc >= 0) & (loc < _WIN)
                    spread = s * 160 + l * 16 + lax.iota(jnp.int32, 16)
                    dst_v[j, sl] = jnp.where(valid, loc, spread)
                    src_v[j, sl] = jnp.where(valid, sv, _ZROW)
                return 0
            lax.fori_loop(0, 80, remap, 0)

            pltpu.async_copy(g_hbm.at[src_v.at[0]], rows_a, sem_a)

            def pair(jj, _):
                j = jj * 2
                pltpu.make_async_copy(g_hbm.at[src_v.at[j]],
                                      rows_a, sem_a).wait()
                pltpu.async_copy(g_hbm.at[src_v.at[j + 1]], rows_b, sem_b)
                pltpu.sync_copy(rows_a, acc.at[dst_v.at[j]], add=True)
                pltpu.make_async_copy(g_hbm.at[src_v.at[j + 1]],
                                      rows_b, sem_b).wait()

                @pl.when(jj < 39)
                def _():
                    pltpu.async_copy(g_hbm.at[src_v.at[j + 2]], rows_a, sem_a)

                pltpu.sync_copy(rows_b, acc.at[dst_v.at[j + 1]], add=True)
                return 0

            lax.fori_loop(0, 40, pair, 0)

        plsc.subcore_barrier()
        pltpu.sync_copy(acc.at[pl.ds(s * 160, 160)],
                        out_hbm.at[pl.ds(base + s * 160, 160)])
        plsc.subcore_barrier()

    @pl.when(c == 0)
    def _():
        do_phase(0)
        do_phase(_WIN)

    @pl.when(c == 1)
    def _():
        do_phase(2 * _WIN)
        do_phase(3 * _WIN)


_edge_pass = functools.partial(
    pl.kernel,
    out_type=jax.ShapeDtypeStruct((_ACC_ROWS, 128), jnp.float32),
    mesh=_sc_mesh,
    scratch_types=[
        pltpu.VMEM((80, 128), jnp.int32),
        pltpu.VMEM((80, 128), jnp.int32),
        pltpu.VMEM((128, 128), jnp.float32),
        pltpu.VMEM((128, 128), jnp.float32),
        pltpu.VMEM((128, 128), jnp.float32),
        pltpu.VMEM_SHARED((_WIN, 128), jnp.float32),
        pltpu.SemaphoreType.DMA,
        pltpu.SemaphoreType.DMA,
    ],
)(_edge_body)


_MWORDS = SP * SP
_MPT = _MWORDS // _NS


def _stats_body(src_hbm, dst_hbm, seg_hbm, deg_hbm, mcnt_hbm,
                src_v, dst_v, code_v, ps_v, pd_v, ones_v, zv,
                sem_g, accdeg, accm):
    c = lax.axis_index("c")
    s = lax.axis_index("s")
    w = c * _NS + s

    def zo(i, _):
        zv[pl.ds(i * 16, 16)] = jnp.zeros((16,), jnp.float32)
        return 0
    lax.fori_loop(0, 128, zo, 0)
    for l in range(8):
        ones_v[pl.ds(l * 16, 16)] = jnp.ones((16,), jnp.float32)

    pltpu.sync_copy(zv.at[pl.ds(0, 640)], accdeg.at[pl.ds(s * 640, 640)])
    for k in range(_MPT // 2048):
        pltpu.sync_copy(zv, accm.at[pl.ds(s * _MPT + k * 2048, 2048)])

    pltpu.sync_copy(src_hbm.at[pl.ds(w * _EROWS, _EROWS)], src_v)
    pltpu.sync_copy(dst_hbm.at[pl.ds(w * _EROWS, _EROWS)], dst_v)
    plsc.subcore_barrier()

    def degj(j, _):
        pltpu.sync_copy(ones_v, accdeg.at[dst_v.at[j]], add=True)
        return 0
    lax.fori_loop(0, _EROWS, degj, 0)

    def codej(j, _):
        pltpu.async_copy(seg_hbm.at[src_v.at[j]], ps_v, sem_g).wait()
        pltpu.async_copy(seg_hbm.at[dst_v.at[j]], pd_v, sem_g).wait()
        for l in range(8):
            ps = ps_v[pl.ds(l * 16, 16)]
            pd = pd_v[pl.ds(l * 16, 16)]
            code_v[j, pl.ds(l * 16, 16)] = ps * SP + pd
        return 0
    lax.fori_loop(0, _EROWS, codej, 0)

    def mj(j, _):
        pltpu.sync_copy(ones_v, accm.at[code_v.at[j]], add=True)
        return 0
    lax.fori_loop(0, _EROWS, mj, 0)

    plsc.subcore_barrier()
    pltpu.sync_copy(accdeg.at[pl.ds(s * 640, 640)],
                    deg_hbm.at[pl.ds(c * _ACC_ROWS + s * 640, 640)])
    pltpu.sync_copy(accm.at[pl.ds(s * _MPT, _MPT)],
                    mcnt_hbm.at[pl.ds(c * _MWORDS + s * _MPT, _MPT)])


_stats_pass = functools.partial(
    pl.kernel,
    out_type=(jax.ShapeDtypeStruct((_NC * _ACC_ROWS,), jnp.float32),
              jax.ShapeDtypeStruct((_NC * _MWORDS,), jnp.float32)),
    mesh=_sc_mesh,
    scratch_types=[
        pltpu.VMEM((_EROWS, 128), jnp.int32),
        pltpu.VMEM((_EROWS, 128), jnp.int32),
        pltpu.VMEM((_EROWS, 128), jnp.int32),
        pltpu.VMEM((128,), jnp.int32),
        pltpu.VMEM((128,), jnp.int32),
        pltpu.VMEM((128,), jnp.float32),
        pltpu.VMEM((2048,), jnp.float32),
        pltpu.SemaphoreType.DMA,
        pltpu.VMEM_SHARED((_ACC_ROWS,), jnp.float32),
        pltpu.VMEM_SHARED((_MWORDS,), jnp.float32),
    ],
)(_stats_body)


def _tcpool_kernel(h_ref, seg_ref, sum_ref, cnt_ref):
    i = pl.program_id(0)
    oh = jnp.where(seg_ref[0, 0, :][:, None] ==
                   lax.broadcasted_iota(jnp.int32, (1000, SP), 1), 1.0, 0.0)
    psum = lax.dot_general(oh, h_ref[...], (((0,), (0,)), ((), ())),
                           preferred_element_type=jnp.float32)
    pcnt = jnp.sum(oh, axis=0, keepdims=True)

    @pl.when(i == 0)
    def _():
        sum_ref[...] = jnp.zeros_like(sum_ref)
        cnt_ref[...] = jnp.zeros_like(cnt_ref)

    sum_ref[...] += psum
    cnt_ref[...] += pcnt


def _tcpool(h, seg):
    return pl.pallas_call(
        _tcpool_kernel,
        grid=(N // 1000,),
        in_specs=[pl.BlockSpec((1000, 128), lambda i: (i, 0)),
                  pl.BlockSpec((1, 1, 1000), lambda i: (i, 0, 0))],
        out_specs=[pl.BlockSpec((SP, 128), lambda i: (0, 0)),
                   pl.BlockSpec((1, SP), lambda i: (0, 0))],
        out_shape=[jax.ShapeDtypeStruct((SP, 128), jnp.float32),
                   jax.ShapeDtypeStruct((1, SP), jnp.float32)],
    )(h, seg.reshape(N // 1000, 1, 1000))


def _elu(x):
    return jnp.where(x > 0, x, jnp.expm1(x))


def _mm_kernel(a_ref, w_ref, b_ref, o_ref):
    o_ref[...] = jnp.dot(a_ref[...], w_ref[...],
                         preferred_element_type=jnp.float32) + b_ref[...]


def _mm(a, w, b):
    m, k = a.shape
    n = w.shape[1]
    blk = 1000
    return pl.pallas_call(
        _mm_kernel,
        grid=(m // blk,),
        in_specs=[pl.BlockSpec((blk, k), lambda i: (i, 0)),
                  pl.BlockSpec((k, n), lambda i: (0, 0)),
                  pl.BlockSpec((1, n), lambda i: (0, 0))],
        out_specs=pl.BlockSpec((blk, n), lambda i: (i, 0)),
        out_shape=jax.ShapeDtypeStruct((m, n), jnp.float32),
    )(a, w, b.reshape(1, -1))


def kernel(x, adj, num_graphs, in_batch, cluster, W1, b1, W2, b2, W3, b3,
           Wt1, bt1, Wt2, bt2, W4, b4, W5, b5, Wf1, bf1, Wf2, bf2, Wf3, bf3):
    src, dst = adj[0], adj[1]
    src2d = jnp.concatenate(
        [src, jnp.zeros((_EPAD - E,), src.dtype)]).reshape(_EPAD // 128, 128)
    dst2d = jnp.concatenate(
        [dst, jnp.full((_EPAD - E,), N, dst.dtype)]).reshape(_EPAD // 128, 128)
    seg_pad = jnp.concatenate(
        [in_batch * CN + cluster,
         jnp.full((_ACC_ROWS - N,), NG * CN, jnp.int32)])

    deg1, mcnt1 = _stats_pass(src2d, dst2d, seg_pad)
    deg2 = deg1.reshape(_NC, _ACC_ROWS)
    mcnt2 = mcnt1.reshape(_NC, _MWORDS)
    deg = deg2[0, :N] + deg2[1, :N] + 1.0
    dinv = 1.0 / jnp.sqrt(deg)

    zrows = jnp.zeros((_ACC_ROWS - N, 128), jnp.float32)

    def conv(u, W, b):
        cm = _mm(u, W, jnp.zeros((128,), jnp.float32))
        g = dinv[:, None] * cm
        sp = _edge_pass(jnp.concatenate([g, zrows]), src2d, dst2d)
        return dinv[:, None] * (sp[:N] + g) + b

    h = _elu(conv(x, W1, b1))
    h = _elu(conv(h, W2, b2))
    h3 = conv(h, W3, b3)

    mean = h3.mean(axis=0)
    var = (h3 * h3).mean(axis=0) - mean * mean
    rstd = 1.0 / jnp.sqrt(var + 1e-5)

    segsum, cnt1 = _tcpool(h3, seg_pad[:N])
    counts = cnt1[0]
    px = (segsum - counts[:, None] * mean[None, :]) \
        / jnp.maximum(counts, 1.0)[:, None] * rstd[None, :]

    mc = (mcnt2[0] + mcnt2[1]).reshape(SP, SP)
    row_ids = lax.broadcasted_iota(jnp.int32, (SP, SP), 0)
    col_ids = lax.broadcasted_iota(jnp.int32, (SP, SP), 1)
    ok = (mc > 0) & (row_ids != col_ids) \
        & (row_ids < NG * CN) & (col_ids < NG * CN)
    M = jnp.where(ok, 1.0, 0.0)
    degc = M.sum(axis=0) + 1.0
    dinvc = 1.0 / jnp.sqrt(degc)

    def cconv(u, W, b):
        v = dinvc[:, None] * (u @ W)
        t = lax.dot_general(M, v, (((0,), (0,)), ((), ())))
        return dinvc[:, None] * (t + v) + b

    z = _elu(px @ Wt1 + bt1)
    z = _elu(z @ Wt2 + bt2)
    z = _elu(cconv(z, W4, b4))
    z = _elu(cconv(z, W5, b5))
    z = z @ Wf1 + bf1
    k = z[:NG * CN].reshape(-1, CN)
    k = _elu(_mm(k, Wf2, bf2))
    k = _mm(k, Wf3, bf3)
    return k


# speedup vs baseline: 27.3764x; 27.3764x over previous
import functools

import jax
import jax.numpy as jnp
from jax import lax
from jax.experimental import pallas as pl
from jax.experimental.pallas import tpu as pltpu
from jax.experimental.pallas import tpu_sc as plsc

N = 10000
E = 320000
CN = 50
NG = 10
SP = 512

_NC, _NS = 2, 16
_NW = _NC * _NS
_EROWS = 80
_EPT = _EROWS * 128
_EPAD = _NW * _EPT
_ACC_ROWS = 10240
_RPT = _ACC_ROWS // _NS

_sc_mesh = plsc.VectorSubcoreMesh(core_axis_name="c", subcore_axis_name="s",
                                  num_cores=_NC, num_subcores=_NS)


def _zero_rows(buf, rows, cols):
    def zrow(i, _):
        for l in range(cols // 16):
            buf[i, pl.ds(l * 16, 16)] = jnp.zeros((16,), jnp.float32)
        return 0
    lax.fori_loop(0, rows, zrow, 0)


_ER = _EPAD // _NS // 128
_WIN = 2560
_ZROW = N


def _edge_body(g_hbm, src_hbm, dst_hbm, out_hbm,
               src_v, dst_v, rows_a, rows_b, zbuf, acc, sem_a, sem_b):
    c = lax.axis_index("c")
    s = lax.axis_index("s")

    _zero_rows(zbuf, 128, 128)

    def do_phase(base):
        pltpu.sync_copy(zbuf, acc.at[pl.ds(s * 160, 128)])
        pltpu.sync_copy(zbuf.at[pl.ds(0, 32)],
                        acc.at[pl.ds(s * 160 + 128, 32)])
        plsc.subcore_barrier()

        for b in range(2):
            pltpu.sync_copy(src_hbm.at[pl.ds(s * _ER + b * 80, 80)], src_v)
            pltpu.sync_copy(dst_hbm.at[pl.ds(s * _ER + b * 80, 80)], dst_v)

            def remap(j, _):
                for l in range(8):
                    sl = pl.ds(l * 16, 16)
                    dv = dst_v[j, sl]
                    sv = src_v[j, sl]
                    loc = dv - base
                    valid = (loc >= 0) & (loc < _WIN)
                    lane = l * 16 + lax.iota(jnp.int32, 16)
                    dst_v[j, sl] = jnp.where(valid, loc, s * 160 + lane)
                    src_v[j, sl] = jnp.where(valid, sv, _ZROW + lane)
                return 0
            lax.fori_loop(0, 80, remap, 0)

            pltpu.async_copy(g_hbm.at[src_v.at[0]], rows_a, sem_a)

            def pair(jj, _):
                j = jj * 2
                pltpu.make_async_copy(g_hbm.at[src_v.at[j]],
                                      rows_a, sem_a).wait()
                pltpu.async_copy(g_hbm.at[src_v.at[j + 1]], rows_b, sem_b)
                pltpu.sync_copy(rows_a, acc.at[dst_v.at[j]], add=True)
                pltpu.make_async_copy(g_hbm.at[src_v.at[j + 1]],
                                      rows_b, sem_b).wait()

                @pl.when(jj < 39)
                def _():
                    pltpu.async_copy(g_hbm.at[src_v.at[j + 2]], rows_a, sem_a)

                pltpu.sync_copy(rows_b, acc.at[dst_v.at[j + 1]], add=True)
                return 0

            lax.fori_loop(0, 40, pair, 0)

        plsc.subcore_barrier()
        pltpu.sync_copy(acc.at[pl.ds(s * 160, 160)],
                        out_hbm.at[pl.ds(base + s * 160, 160)])
        plsc.subcore_barrier()

    @pl.when(c == 0)
    def _():
        do_phase(0)
        do_phase(_WIN)

    @pl.when(c == 1)
    def _():
        do_phase(2 * _WIN)
        do_phase(3 * _WIN)


_edge_pass = functools.partial(
    pl.kernel,
    out_type=jax.ShapeDtypeStruct((_ACC_ROWS, 128), jnp.float32),
    mesh=_sc_mesh,
    scratch_types=[
        pltpu.VMEM((80, 128), jnp.int32),
        pltpu.VMEM((80, 128), jnp.int32),
        pltpu.VMEM((128, 128), jnp.float32),
        pltpu.VMEM((128, 128), jnp.float32),
        pltpu.VMEM((128, 128), jnp.float32),
        pltpu.VMEM_SHARED((_WIN, 128), jnp.float32),
        pltpu.SemaphoreType.DMA,
        pltpu.SemaphoreType.DMA,
    ],
)(_edge_body)


_MWORDS = SP * SP
_MPT = _MWORDS // _NS


def _stats_body(src_hbm, dst_hbm, seg_hbm, deg_hbm, mcnt_hbm,
                src_v, dst_v, code_v, ps_v, pd_v, ones_v, zv,
                sem_g, accdeg, accm):
    c = lax.axis_index("c")
    s = lax.axis_index("s")
    w = c * _NS + s

    def zo(i, _):
        zv[pl.ds(i * 16, 16)] = jnp.zeros((16,), jnp.float32)
        return 0
    lax.fori_loop(0, 128, zo, 0)
    for l in range(8):
        ones_v[pl.ds(l * 16, 16)] = jnp.ones((16,), jnp.float32)

    pltpu.sync_copy(zv.at[pl.ds(0, 640)], accdeg.at[pl.ds(s * 640, 640)])
    for k in range(_MPT // 2048):
        pltpu.sync_copy(zv, accm.at[pl.ds(s * _MPT + k * 2048, 2048)])

    pltpu.sync_copy(src_hbm.at[pl.ds(w * _EROWS, _EROWS)], src_v)
    pltpu.sync_copy(dst_hbm.at[pl.ds(w * _EROWS, _EROWS)], dst_v)
    plsc.subcore_barrier()

    def degj(j, _):
        pltpu.sync_copy(ones_v, accdeg.at[dst_v.at[j]], add=True)
        return 0
    lax.fori_loop(0, _EROWS, degj, 0)

    def codej(j, _):
        pltpu.async_copy(seg_hbm.at[src_v.at[j]], ps_v, sem_g).wait()
        pltpu.async_copy(seg_hbm.at[dst_v.at[j]], pd_v, sem_g).wait()
        for l in range(8):
            ps = ps_v[pl.ds(l * 16, 16)]
            pd = pd_v[pl.ds(l * 16, 16)]
            code_v[j, pl.ds(l * 16, 16)] = ps * SP + pd
        return 0
    lax.fori_loop(0, _EROWS, codej, 0)

    def mj(j, _):
        pltpu.sync_copy(ones_v, accm.at[code_v.at[j]], add=True)
        return 0
    lax.fori_loop(0, _EROWS, mj, 0)

    plsc.subcore_barrier()
    pltpu.sync_copy(accdeg.at[pl.ds(s * 640, 640)],
                    deg_hbm.at[pl.ds(c * _ACC_ROWS + s * 640, 640)])
    pltpu.sync_copy(accm.at[pl.ds(s * _MPT, _MPT)],
                    mcnt_hbm.at[pl.ds(c * _MWORDS + s * _MPT, _MPT)])


_stats_pass = functools.partial(
    pl.kernel,
    out_type=(jax.ShapeDtypeStruct((_NC * _ACC_ROWS,), jnp.float32),
              jax.ShapeDtypeStruct((_NC * _MWORDS,), jnp.float32)),
    mesh=_sc_mesh,
    scratch_types=[
        pltpu.VMEM((_EROWS, 128), jnp.int32),
        pltpu.VMEM((_EROWS, 128), jnp.int32),
        pltpu.VMEM((_EROWS, 128), jnp.int32),
        pltpu.VMEM((128,), jnp.int32),
        pltpu.VMEM((128,), jnp.int32),
        pltpu.VMEM((128,), jnp.float32),
        pltpu.VMEM((2048,), jnp.float32),
        pltpu.SemaphoreType.DMA,
        pltpu.VMEM_SHARED((_ACC_ROWS,), jnp.float32),
        pltpu.VMEM_SHARED((_MWORDS,), jnp.float32),
    ],
)(_stats_body)


def _tcpool_kernel(h_ref, seg_ref, sum_ref, cnt_ref):
    i = pl.program_id(0)
    oh = jnp.where(seg_ref[0, 0, :][:, None] ==
                   lax.broadcasted_iota(jnp.int32, (1000, SP), 1), 1.0, 0.0)
    psum = lax.dot_general(oh, h_ref[...], (((0,), (0,)), ((), ())),
                           preferred_element_type=jnp.float32)
    pcnt = jnp.sum(oh, axis=0, keepdims=True)

    @pl.when(i == 0)
    def _():
        sum_ref[...] = jnp.zeros_like(sum_ref)
        cnt_ref[...] = jnp.zeros_like(cnt_ref)

    sum_ref[...] += psum
    cnt_ref[...] += pcnt


def _tcpool(h, seg):
    return pl.pallas_call(
        _tcpool_kernel,
        grid=(N // 1000,),
        in_specs=[pl.BlockSpec((1000, 128), lambda i: (i, 0)),
                  pl.BlockSpec((1, 1, 1000), lambda i: (i, 0, 0))],
        out_specs=[pl.BlockSpec((SP, 128), lambda i: (0, 0)),
                   pl.BlockSpec((1, SP), lambda i: (0, 0))],
        out_shape=[jax.ShapeDtypeStruct((SP, 128), jnp.float32),
                   jax.ShapeDtypeStruct((1, SP), jnp.float32)],
    )(h, seg.reshape(N // 1000, 1, 1000))


def _elu(x):
    return jnp.where(x > 0, x, jnp.expm1(x))


def _mm_kernel(a_ref, w_ref, b_ref, o_ref):
    o_ref[...] = jnp.dot(a_ref[...], w_ref[...],
                         preferred_element_type=jnp.float32) + b_ref[...]


def _mm(a, w, b):
    m, k = a.shape
    n = w.shape[1]
    blk = 1000
    return pl.pallas_call(
        _mm_kernel,
        grid=(m // blk,),
        in_specs=[pl.BlockSpec((blk, k), lambda i: (i, 0)),
                  pl.BlockSpec((k, n), lambda i: (0, 0)),
                  pl.BlockSpec((1, n), lambda i: (0, 0))],
        out_specs=pl.BlockSpec((blk, n), lambda i: (i, 0)),
        out_shape=jax.ShapeDtypeStruct((m, n), jnp.float32),
    )(a, w, b.reshape(1, -1))


def kernel(x, adj, num_graphs, in_batch, cluster, W1, b1, W2, b2, W3, b3,
           Wt1, bt1, Wt2, bt2, W4, b4, W5, b5, Wf1, bf1, Wf2, bf2, Wf3, bf3):
    src, dst = adj[0], adj[1]
    src2d = jnp.concatenate(
        [src, jnp.zeros((_EPAD - E,), src.dtype)]).reshape(_EPAD // 128, 128)
    dst2d = jnp.concatenate(
        [dst, jnp.full((_EPAD - E,), N, dst.dtype)]).reshape(_EPAD // 128, 128)
    seg_pad = jnp.concatenate(
        [in_batch * CN + cluster,
         jnp.full((_ACC_ROWS - N,), NG * CN, jnp.int32)])

    deg1, mcnt1 = _stats_pass(src2d, dst2d, seg_pad)
    deg2 = deg1.reshape(_NC, _ACC_ROWS)
    mcnt2 = mcnt1.reshape(_NC, _MWORDS)
    deg = deg2[0, :N] + deg2[1, :N] + 1.0
    dinv = 1.0 / jnp.sqrt(deg)

    zrows = jnp.zeros((_ACC_ROWS - N, 128), jnp.float32)

    def conv(u, W, b):
        cm = _mm(u, W, jnp.zeros((128,), jnp.float32))
        g = dinv[:, None] * cm
        sp = _edge_pass(jnp.concatenate([g, zrows]), src2d, dst2d)
        return dinv[:, None] * (sp[:N] + g) + b

    h = _elu(conv(x, W1, b1))
    h = _elu(conv(h, W2, b2))
    h3 = conv(h, W3, b3)

    mean = h3.mean(axis=0)
    var = (h3 * h3).mean(axis=0) - mean * mean
    rstd = 1.0 / jnp.sqrt(var + 1e-5)

    segsum, cnt1 = _tcpool(h3, seg_pad[:N])
    counts = cnt1[0]
    px = (segsum - counts[:, None] * mean[None, :]) \
        / jnp.maximum(counts, 1.0)[:, None] * rstd[None, :]

    mc = (mcnt2[0] + mcnt2[1]).reshape(SP, SP)
    row_ids = lax.broadcasted_iota(jnp.int32, (SP, SP), 0)
    col_ids = lax.broadcasted_iota(jnp.int32, (SP, SP), 1)
    ok = (mc > 0) & (row_ids != col_ids) \
        & (row_ids < NG * CN) & (col_ids < NG * CN)
    M = jnp.where(ok, 1.0, 0.0)
    degc = M.sum(axis=0) + 1.0
    dinvc = 1.0 / jnp.sqrt(degc)

    def cconv(u, W, b):
        v = dinvc[:, None] * (u @ W)
        t = lax.dot_general(M, v, (((0,), (0,)), ((), ())))
        return dinvc[:, None] * (t + v) + b

    z = _elu(px @ Wt1 + bt1)
    z = _elu(z @ Wt2 + bt2)
    z = _elu(cconv(z, W4, b4))
    z = _elu(cconv(z, W5, b5))
    z = z @ Wf1 + bf1
    k = z[:NG * CN].reshape(-1, CN)
    k = _elu(_mm(k, Wf2, bf2))
    k = _mm(k, Wf3, bf3)
    return k


# speedup vs baseline: 33.4359x; 1.2213x over previous
import functools

import jax
import jax.numpy as jnp
from jax import lax
from jax.experimental import pallas as pl
from jax.experimental.pallas import tpu as pltpu
from jax.experimental.pallas import tpu_sc as plsc

N = 10000
E = 320000
CN = 50
NG = 10
SP = 512

_NC, _NS = 2, 16
_NW = _NC * _NS
_EROWS = 80
_EPT = _EROWS * 128
_EPAD = _NW * _EPT
_ACC_ROWS = 10240
_RPT = _ACC_ROWS // _NS

_sc_mesh = plsc.VectorSubcoreMesh(core_axis_name="c", subcore_axis_name="s",
                                  num_cores=_NC, num_subcores=_NS)


def _zero_rows(buf, rows, cols):
    def zrow(i, _):
        for l in range(cols // 16):
            buf[i, pl.ds(l * 16, 16)] = jnp.zeros((16,), jnp.float32)
        return 0
    lax.fori_loop(0, rows, zrow, 0)


_ER = _EPAD // _NS // 128
_WIN = 2560
_ZROW = N


def _edge_body(g_hbm, src_hbm, dst_hbm, out_hbm,
               src_v, dst_v, rows_a, rows_b, zbuf, acc, sem_a, sem_b):
    c = lax.axis_index("c")
    s = lax.axis_index("s")

    _zero_rows(zbuf, 128, 128)

    def do_phase(base):
        pltpu.sync_copy(zbuf, acc.at[pl.ds(s * 160, 128)])
        pltpu.sync_copy(zbuf.at[pl.ds(0, 32)],
                        acc.at[pl.ds(s * 160 + 128, 32)])
        plsc.subcore_barrier()

        for b in range(2):
            pltpu.sync_copy(src_hbm.at[pl.ds(s * _ER + b * 80, 80)], src_v)
            pltpu.sync_copy(dst_hbm.at[pl.ds(s * _ER + b * 80, 80)], dst_v)

            def remap(j, _):
                for l in range(8):
                    sl = pl.ds(l * 16, 16)
                    dv = dst_v[j, sl]
                    sv = src_v[j, sl]
                    loc = dv - base
                    valid = (loc >= 0) & (loc < _WIN)
                    lane = l * 16 + lax.iota(jnp.int32, 16)
                    dst_v[j, sl] = jnp.where(valid, loc, s * 160 + lane)
                    src_v[j, sl] = jnp.where(valid, sv, _ZROW + lane)
                return 0
            lax.fori_loop(0, 80, remap, 0)

            pltpu.async_copy(g_hbm.at[src_v.at[0]], rows_a, sem_a)

            def pair(jj, _):
                j = jj * 2
                pltpu.make_async_copy(g_hbm.at[src_v.at[j]],
                                      rows_a, sem_a).wait()
                pltpu.async_copy(g_hbm.at[src_v.at[j + 1]], rows_b, sem_b)
                pltpu.sync_copy(rows_a, acc.at[dst_v.at[j]], add=True)
                pltpu.make_async_copy(g_hbm.at[src_v.at[j + 1]],
                                      rows_b, sem_b).wait()

                @pl.when(jj < 39)
                def _():
                    pltpu.async_copy(g_hbm.at[src_v.at[j + 2]], rows_a, sem_a)

                pltpu.sync_copy(rows_b, acc.at[dst_v.at[j + 1]], add=True)
                return 0

            lax.fori_loop(0, 40, pair, 0)

        plsc.subcore_barrier()
        pltpu.sync_copy(acc.at[pl.ds(s * 160, 160)],
                        out_hbm.at[pl.ds(base + s * 160, 160)])
        plsc.subcore_barrier()

    @pl.when(c == 0)
    def _():
        do_phase(0)
        do_phase(_WIN)

    @pl.when(c == 1)
    def _():
        do_phase(2 * _WIN)
        do_phase(3 * _WIN)


_edge_pass = functools.partial(
    pl.kernel,
    out_type=jax.ShapeDtypeStruct((_ACC_ROWS, 128), jnp.float32),
    mesh=_sc_mesh,
    scratch_types=[
        pltpu.VMEM((80, 128), jnp.int32),
        pltpu.VMEM((80, 128), jnp.int32),
        pltpu.VMEM((128, 128), jnp.float32),
        pltpu.VMEM((128, 128), jnp.float32),
        pltpu.VMEM((128, 128), jnp.float32),
        pltpu.VMEM_SHARED((_WIN, 128), jnp.float32),
        pltpu.SemaphoreType.DMA,
        pltpu.SemaphoreType.DMA,
    ],
)(_edge_body)


_MWORDS = SP * SP
_MPT = _MWORDS // _NS


def _stats_body(src_hbm, dst_hbm, seg_hbm, deg_hbm, mcnt_hbm,
                src_v, dst_v, code_v, ps_v, pd_v, ones_v, zv,
                sem_g, accdeg, accm):
    c = lax.axis_index("c")
    s = lax.axis_index("s")
    w = c * _NS + s

    def zo(i, _):
        zv[pl.ds(i * 16, 16)] = jnp.zeros((16,), jnp.float32)
        return 0
    lax.fori_loop(0, 128, zo, 0)
    for l in range(8):
        ones_v[pl.ds(l * 16, 16)] = jnp.ones((16,), jnp.float32)

    pltpu.sync_copy(zv.at[pl.ds(0, 640)], accdeg.at[pl.ds(s * 640, 640)])
    for k in range(_MPT // 2048):
        pltpu.sync_copy(zv, accm.at[pl.ds(s * _MPT + k * 2048, 2048)])

    pltpu.sync_copy(src_hbm.at[pl.ds(w * _EROWS, _EROWS)], src_v)
    pltpu.sync_copy(dst_hbm.at[pl.ds(w * _EROWS, _EROWS)], dst_v)
    plsc.subcore_barrier()

    def degj(j, _):
        pltpu.sync_copy(ones_v, accdeg.at[dst_v.at[j]], add=True)
        return 0
    lax.fori_loop(0, _EROWS, degj, 0)

    def codej(j, _):
        pltpu.async_copy(seg_hbm.at[src_v.at[j]], ps_v, sem_g).wait()
        pltpu.async_copy(seg_hbm.at[dst_v.at[j]], pd_v, sem_g).wait()
        for l in range(8):
            ps = ps_v[pl.ds(l * 16, 16)]
            pd = pd_v[pl.ds(l * 16, 16)]
            code_v[j, pl.ds(l * 16, 16)] = ps * SP + pd
        return 0
    lax.fori_loop(0, _EROWS, codej, 0)

    def mj(j, _):
        pltpu.sync_copy(ones_v, accm.at[code_v.at[j]], add=True)
        return 0
    lax.fori_loop(0, _EROWS, mj, 0)

    plsc.subcore_barrier()
    pltpu.sync_copy(accdeg.at[pl.ds(s * 640, 640)],
                    deg_hbm.at[pl.ds(c * _ACC_ROWS + s * 640, 640)])
    pltpu.sync_copy(accm.at[pl.ds(s * _MPT, _MPT)],
                    mcnt_hbm.at[pl.ds(c * _MWORDS + s * _MPT, _MPT)])


_stats_pass = functools.partial(
    pl.kernel,
    out_type=(jax.ShapeDtypeStruct((_NC * _ACC_ROWS,), jnp.float32),
              jax.ShapeDtypeStruct((_NC * _MWORDS,), jnp.float32)),
    mesh=_sc_mesh,
    scratch_types=[
        pltpu.VMEM((_EROWS, 128), jnp.int32),
        pltpu.VMEM((_EROWS, 128), jnp.int32),
        pltpu.VMEM((_EROWS, 128), jnp.int32),
        pltpu.VMEM((128,), jnp.int32),
        pltpu.VMEM((128,), jnp.int32),
        pltpu.VMEM((128,), jnp.float32),
        pltpu.VMEM((2048,), jnp.float32),
        pltpu.SemaphoreType.DMA,
        pltpu.VMEM_SHARED((_ACC_ROWS,), jnp.float32),
        pltpu.VMEM_SHARED((_MWORDS,), jnp.float32),
    ],
)(_stats_body)


def _tcpool_kernel(h_ref, seg_ref, sum_ref, cnt_ref, sq_ref):
    i = pl.program_id(0)
    h = h_ref[...]
    oh = jnp.where(seg_ref[0, 0, :][:, None] ==
                   lax.broadcasted_iota(jnp.int32, (1000, SP), 1), 1.0, 0.0)
    psum = lax.dot_general(oh, h, (((0,), (0,)), ((), ())),
                           preferred_element_type=jnp.float32)
    pcnt = jnp.sum(oh, axis=0, keepdims=True)

    @pl.when(i == 0)
    def _():
        sum_ref[...] = jnp.zeros_like(sum_ref)
        cnt_ref[...] = jnp.zeros_like(cnt_ref)
        sq_ref[...] = jnp.zeros_like(sq_ref)

    sum_ref[...] += psum
    cnt_ref[...] += pcnt
    sq_ref[...] += jnp.sum(h * h, axis=0, keepdims=True)


def _tcpool(h, seg):
    return pl.pallas_call(
        _tcpool_kernel,
        grid=(N // 1000,),
        in_specs=[pl.BlockSpec((1000, 128), lambda i: (i, 0)),
                  pl.BlockSpec((1, 1, 1000), lambda i: (i, 0, 0))],
        out_specs=[pl.BlockSpec((SP, 128), lambda i: (0, 0)),
                   pl.BlockSpec((1, SP), lambda i: (0, 0)),
                   pl.BlockSpec((1, 128), lambda i: (0, 0))],
        out_shape=[jax.ShapeDtypeStruct((SP, 128), jnp.float32),
                   jax.ShapeDtypeStruct((1, SP), jnp.float32),
                   jax.ShapeDtypeStruct((1, 128), jnp.float32)],
    )(h, seg.reshape(N // 1000, 1, 1000))


def _pelu(x):
    return jnp.where(x > 0, x, jnp.exp(x) - 1.0)


def _coarse_kernel(segsum_ref, cnt_ref, sq_ref, mc_ref,
                   wt1_ref, bt1_ref, wt2_ref, bt2_ref,
                   w4_ref, b4_ref, w5_ref, b5_ref, wf1_ref, bf1_ref, o_ref):
    mean = jnp.sum(segsum_ref[...], axis=0, keepdims=True) / N
    var = sq_ref[...] / N - mean * mean
    rstd = lax.rsqrt(var + 1e-5)
    cnt = cnt_ref[0, :][:, None]
    px = (segsum_ref[...] - cnt * mean) / jnp.maximum(cnt, 1.0) * rstd

    mc = mc_ref[0] + mc_ref[1]
    rid = lax.broadcasted_iota(jnp.int32, (SP, SP), 0)
    cid = lax.broadcasted_iota(jnp.int32, (SP, SP), 1)
    keep = (mc > 0) & (rid != cid) & (rid < NG * CN) & (cid < NG * CN)
    M = jnp.where(keep, 1.0, 0.0)
    degc = jnp.sum(M, axis=0, keepdims=True) + 1.0
    dinvc = lax.rsqrt(degc).reshape(SP, 1)

    def cconv(u, W, b):
        v = dinvc * jnp.dot(u, W, preferred_element_type=jnp.float32)
        t = lax.dot_general(M, v, (((0,), (0,)), ((), ())),
                            preferred_element_type=jnp.float32)
        return dinvc * (t + v) + b

    z = _pelu(jnp.dot(px, wt1_ref[...],
                      preferred_element_type=jnp.float32) + bt1_ref[...])
    z = _pelu(jnp.dot(z, wt2_ref[...],
                      preferred_element_type=jnp.float32) + bt2_ref[...])
    z = _pelu(cconv(z, w4_ref[...], b4_ref[...]))
    z = _pelu(cconv(z, w5_ref[...], b5_ref[...]))
    o_ref[...] = jnp.dot(z, wf1_ref[...],
                         preferred_element_type=jnp.float32) + bf1_ref[...]


def _coarse(segsum, cnt, sq, mc2, Wt1, bt1, Wt2, bt2, W4, b4, W5, b5,
            Wf1, bf1):
    return pl.pallas_call(
        _coarse_kernel,
        out_shape=jax.ShapeDtypeStruct((SP, 1000), jnp.float32),
    )(segsum, cnt, sq, mc2, Wt1, bt1.reshape(1, 32), Wt2,
      bt2.reshape(1, 128), W4, b4.reshape(1, 128), W5, b5.reshape(1, 128),
      Wf1, bf1.reshape(1, 1000))


def _elu(x):
    return jnp.where(x > 0, x, jnp.expm1(x))


def _mm_kernel(a_ref, w_ref, b_ref, o_ref):
    o_ref[...] = jnp.dot(a_ref[...], w_ref[...],
                         preferred_element_type=jnp.float32) + b_ref[...]


def _mm(a, w, b):
    m, k = a.shape
    n = w.shape[1]
    blk = 1000
    return pl.pallas_call(
        _mm_kernel,
        grid=(m // blk,),
        in_specs=[pl.BlockSpec((blk, k), lambda i: (i, 0)),
                  pl.BlockSpec((k, n), lambda i: (0, 0)),
                  pl.BlockSpec((1, n), lambda i: (0, 0))],
        out_specs=pl.BlockSpec((blk, n), lambda i: (i, 0)),
        out_shape=jax.ShapeDtypeStruct((m, n), jnp.float32),
    )(a, w, b.reshape(1, -1))


def kernel(x, adj, num_graphs, in_batch, cluster, W1, b1, W2, b2, W3, b3,
           Wt1, bt1, Wt2, bt2, W4, b4, W5, b5, Wf1, bf1, Wf2, bf2, Wf3, bf3):
    src, dst = adj[0], adj[1]
    src2d = jnp.concatenate(
        [src, jnp.zeros((_EPAD - E,), src.dtype)]).reshape(_EPAD // 128, 128)
    dst2d = jnp.concatenate(
        [dst, jnp.full((_EPAD - E,), N, dst.dtype)]).reshape(_EPAD // 128, 128)
    seg_pad = jnp.concatenate(
        [in_batch * CN + cluster,
         jnp.full((_ACC_ROWS - N,), NG * CN, jnp.int32)])

    deg1, mcnt1 = _stats_pass(src2d, dst2d, seg_pad)
    deg2 = deg1.reshape(_NC, _ACC_ROWS)
    mcnt2 = mcnt1.reshape(_NC, _MWORDS)
    deg = deg2[0, :N] + deg2[1, :N] + 1.0
    dinv = 1.0 / jnp.sqrt(deg)

    zrows = jnp.zeros((_ACC_ROWS - N, 128), jnp.float32)

    def conv(u, W, b):
        cm = _mm(u, W, jnp.zeros((128,), jnp.float32))
        g = dinv[:, None] * cm
        sp = _edge_pass(jnp.concatenate([g, zrows]), src2d, dst2d)
        return dinv[:, None] * (sp[:N] + g) + b

    h = _elu(conv(x, W1, b1))
    h = _elu(conv(h, W2, b2))
    h3 = conv(h, W3, b3)

    segsum, cnt1, sq1 = _tcpool(h3, seg_pad[:N])
    z = _coarse(segsum, cnt1, sq1, mcnt2.reshape(_NC, SP, SP),
                Wt1, bt1, Wt2, bt2, W4, b4, W5, b5, Wf1, bf1)
    k = z[:NG * CN].reshape(-1, CN)
    k = _elu(_mm(k, Wf2, bf2))
    k = _mm(k, Wf3, bf3)
    return k


# speedup vs baseline: 35.2419x; 1.0540x over previous
import functools

import jax
import jax.numpy as jnp
from jax import lax
from jax.experimental import pallas as pl
from jax.experimental.pallas import tpu as pltpu
from jax.experimental.pallas import tpu_sc as plsc

N = 10000
E = 320000
CN = 50
NG = 10
SP = 512

_NC, _NS = 2, 16
_NW = _NC * _NS
_EROWS = 80
_EPT = _EROWS * 128
_EPAD = _NW * _EPT
_ACC_ROWS = 10240
_RPT = _ACC_ROWS // _NS

_sc_mesh = plsc.VectorSubcoreMesh(core_axis_name="c", subcore_axis_name="s",
                                  num_cores=_NC, num_subcores=_NS)


def _zero_rows(buf, rows, cols):
    def zrow(i, _):
        for l in range(cols // 16):
            buf[i, pl.ds(l * 16, 16)] = jnp.zeros((16,), jnp.float32)
        return 0
    lax.fori_loop(0, rows, zrow, 0)


_ER = _EPAD // _NS // 128
_WIN = 2560
_ZROW = N


def _edge_body(g_hbm, src_hbm, dst_hbm, out_hbm,
               src_v, dst_v, rows_a, rows_b, zbuf, acc, sem_a, sem_b):
    c = lax.axis_index("c")
    s = lax.axis_index("s")

    _zero_rows(zbuf, 128, 128)

    def do_phase(base):
        pltpu.sync_copy(zbuf, acc.at[pl.ds(s * 160, 128)])
        pltpu.sync_copy(zbuf.at[pl.ds(0, 32)],
                        acc.at[pl.ds(s * 160 + 128, 32)])
        plsc.subcore_barrier()

        for b in range(2):
            pltpu.sync_copy(src_hbm.at[pl.ds(s * _ER + b * 80, 80)], src_v)
            pltpu.sync_copy(dst_hbm.at[pl.ds(s * _ER + b * 80, 80)], dst_v)

            def remap(j, _):
                for l in range(8):
                    sl = pl.ds(l * 16, 16)
                    dv = dst_v[j, sl]
                    sv = src_v[j, sl]
                    loc = dv - base
                    valid = (loc >= 0) & (loc < _WIN)
                    lane = l * 16 + lax.iota(jnp.int32, 16)
                    dst_v[j, sl] = jnp.where(valid, loc, s * 160 + lane)
                    src_v[j, sl] = jnp.where(valid, sv, _ZROW + lane)
                return 0
            lax.fori_loop(0, 80, remap, 0)

            pltpu.async_copy(g_hbm.at[src_v.at[0]], rows_a, sem_a)

            def pair(jj, _):
                j = jj * 2
                pltpu.make_async_copy(g_hbm.at[src_v.at[j]],
                                      rows_a, sem_a).wait()
                pltpu.async_copy(g_hbm.at[src_v.at[j + 1]], rows_b, sem_b)
                pltpu.sync_copy(rows_a, acc.at[dst_v.at[j]], add=True)
                pltpu.make_async_copy(g_hbm.at[src_v.at[j + 1]],
                                      rows_b, sem_b).wait()

                @pl.when(jj < 39)
                def _():
                    pltpu.async_copy(g_hbm.at[src_v.at[j + 2]], rows_a, sem_a)

                pltpu.sync_copy(rows_b, acc.at[dst_v.at[j + 1]], add=True)
                return 0

            lax.fori_loop(0, 40, pair, 0)

        plsc.subcore_barrier()
        pltpu.sync_copy(acc.at[pl.ds(s * 160, 160)],
                        out_hbm.at[pl.ds(base + s * 160, 160)])
        plsc.subcore_barrier()

    @pl.when(c == 0)
    def _():
        do_phase(0)
        do_phase(_WIN)

    @pl.when(c == 1)
    def _():
        do_phase(2 * _WIN)
        do_phase(3 * _WIN)


_edge_pass = functools.partial(
    pl.kernel,
    out_type=jax.ShapeDtypeStruct((_ACC_ROWS, 128), jnp.float32),
    mesh=_sc_mesh,
    scratch_types=[
        pltpu.VMEM((80, 128), jnp.int32),
        pltpu.VMEM((80, 128), jnp.int32),
        pltpu.VMEM((128, 128), jnp.float32),
        pltpu.VMEM((128, 128), jnp.float32),
        pltpu.VMEM((128, 128), jnp.float32),
        pltpu.VMEM_SHARED((_WIN, 128), jnp.float32),
        pltpu.SemaphoreType.DMA,
        pltpu.SemaphoreType.DMA,
    ],
)(_edge_body)


_MWORDS = SP * SP
_MPT = _MWORDS // _NS


def _stats_body(src_hbm, dst_hbm, seg_hbm, deg_hbm, mcnt_hbm,
                src_v, dst_v, code_v, ps_v, pd_v, ones_v, zv,
                sem_g, accdeg, accm):
    c = lax.axis_index("c")
    s = lax.axis_index("s")
    w = c * _NS + s

    def zo(i, _):
        zv[pl.ds(i * 16, 16)] = jnp.zeros((16,), jnp.float32)
        return 0
    lax.fori_loop(0, 128, zo, 0)
    for l in range(8):
        ones_v[pl.ds(l * 16, 16)] = jnp.ones((16,), jnp.float32)

    pltpu.sync_copy(zv.at[pl.ds(0, 640)], accdeg.at[pl.ds(s * 640, 640)])
    for k in range(_MPT // 2048):
        pltpu.sync_copy(zv, accm.at[pl.ds(s * _MPT + k * 2048, 2048)])

    pltpu.sync_copy(src_hbm.at[pl.ds(w * _EROWS, _EROWS)], src_v)
    pltpu.sync_copy(dst_hbm.at[pl.ds(w * _EROWS, _EROWS)], dst_v)
    plsc.subcore_barrier()

    def degj(j, _):
        pltpu.sync_copy(ones_v, accdeg.at[dst_v.at[j]], add=True)
        return 0
    lax.fori_loop(0, _EROWS, degj, 0)

    def codej(j, _):
        pltpu.async_copy(seg_hbm.at[src_v.at[j]], ps_v, sem_g).wait()
        pltpu.async_copy(seg_hbm.at[dst_v.at[j]], pd_v, sem_g).wait()
        for l in range(8):
            ps = ps_v[pl.ds(l * 16, 16)]
            pd = pd_v[pl.ds(l * 16, 16)]
            code_v[j, pl.ds(l * 16, 16)] = ps * SP + pd
        return 0
    lax.fori_loop(0, _EROWS, codej, 0)

    def mj(j, _):
        pltpu.sync_copy(ones_v, accm.at[code_v.at[j]], add=True)
        return 0
    lax.fori_loop(0, _EROWS, mj, 0)

    plsc.subcore_barrier()
    pltpu.sync_copy(accdeg.at[pl.ds(s * 640, 640)],
                    deg_hbm.at[pl.ds(c * _ACC_ROWS + s * 640, 640)])
    pltpu.sync_copy(accm.at[pl.ds(s * _MPT, _MPT)],
                    mcnt_hbm.at[pl.ds(c * _MWORDS + s * _MPT, _MPT)])


_stats_pass = functools.partial(
    pl.kernel,
    out_type=(jax.ShapeDtypeStruct((_NC * _ACC_ROWS,), jnp.float32),
              jax.ShapeDtypeStruct((_NC * _MWORDS,), jnp.float32)),
    mesh=_sc_mesh,
    scratch_types=[
        pltpu.VMEM((_EROWS, 128), jnp.int32),
        pltpu.VMEM((_EROWS, 128), jnp.int32),
        pltpu.VMEM((_EROWS, 128), jnp.int32),
        pltpu.VMEM((128,), jnp.int32),
        pltpu.VMEM((128,), jnp.int32),
        pltpu.VMEM((128,), jnp.float32),
        pltpu.VMEM((2048,), jnp.float32),
        pltpu.SemaphoreType.DMA,
        pltpu.VMEM_SHARED((_ACC_ROWS,), jnp.float32),
        pltpu.VMEM_SHARED((_MWORDS,), jnp.float32),
    ],
)(_stats_body)


def _tcpool_kernel(sp_ref, g_ref, d_ref, b_ref, seg_ref,
                   sum_ref, cnt_ref, sq_ref):
    i = pl.program_id(0)
    h = lax.rsqrt(d_ref[0, 0, :] + d_ref[0, 1, :] + 1.0)[:, None] \
        * (sp_ref[...] + g_ref[...]) + b_ref[...]
    oh = jnp.where(seg_ref[0, 0, :][:, None] ==
                   lax.broadcasted_iota(jnp.int32, (1000, SP), 1), 1.0, 0.0)
    psum = lax.dot_general(oh, h, (((0,), (0,)), ((), ())),
                           preferred_element_type=jnp.float32)
    pcnt = jnp.sum(oh, axis=0, keepdims=True)

    @pl.when(i == 0)
    def _():
        sum_ref[...] = jnp.zeros_like(sum_ref)
        cnt_ref[...] = jnp.zeros_like(cnt_ref)
        sq_ref[...] = jnp.zeros_like(sq_ref)

    sum_ref[...] += psum
    cnt_ref[...] += pcnt
    sq_ref[...] += jnp.sum(h * h, axis=0, keepdims=True)


def _tcpool(sp, g, deg2, b, seg):
    return pl.pallas_call(
        _tcpool_kernel,
        grid=(N // 1000,),
        in_specs=[pl.BlockSpec((1000, 128), lambda i: (i, 0)),
                  pl.BlockSpec((1000, 128), lambda i: (i, 0)),
                  pl.BlockSpec((1, 2, 1000), lambda i: (i, 0, 0)),
                  pl.BlockSpec((1, 128), lambda i: (0, 0)),
                  pl.BlockSpec((1, 1, 1000), lambda i: (i, 0, 0))],
        out_specs=[pl.BlockSpec((SP, 128), lambda i: (0, 0)),
                   pl.BlockSpec((1, SP), lambda i: (0, 0)),
                   pl.BlockSpec((1, 128), lambda i: (0, 0))],
        out_shape=[jax.ShapeDtypeStruct((SP, 128), jnp.float32),
                   jax.ShapeDtypeStruct((1, SP), jnp.float32),
                   jax.ShapeDtypeStruct((1, 128), jnp.float32)],
    )(sp, g, deg2[:, :N].reshape(2, N // 1000, 1000).transpose(1, 0, 2),
      b.reshape(1, 128), seg.reshape(N // 1000, 1, 1000))


def _pelu(x):
    return jnp.where(x > 0, x, jnp.exp(x) - 1.0)


def _coarse_kernel(segsum_ref, cnt_ref, sq_ref, mc_ref,
                   wt1_ref, bt1_ref, wt2_ref, bt2_ref,
                   w4_ref, b4_ref, w5_ref, b5_ref, wf1_ref, bf1_ref, o_ref):
    mean = jnp.sum(segsum_ref[...], axis=0, keepdims=True) / N
    var = sq_ref[...] / N - mean * mean
    rstd = lax.rsqrt(var + 1e-5)
    cnt = cnt_ref[0, :][:, None]
    px = (segsum_ref[...] - cnt * mean) / jnp.maximum(cnt, 1.0) * rstd

    mc = mc_ref[0] + mc_ref[1]
    rid = lax.broadcasted_iota(jnp.int32, (SP, SP), 0)
    cid = lax.broadcasted_iota(jnp.int32, (SP, SP), 1)
    keep = (mc > 0) & (rid != cid) & (rid < NG * CN) & (cid < NG * CN)
    M = jnp.where(keep, 1.0, 0.0)
    degc = jnp.sum(M, axis=0, keepdims=True) + 1.0
    dinvc = lax.rsqrt(degc).reshape(SP, 1)

    def cconv(u, W, b):
        v = dinvc * jnp.dot(u, W, preferred_element_type=jnp.float32)
        t = lax.dot_general(M, v, (((0,), (0,)), ((), ())),
                            preferred_element_type=jnp.float32)
        return dinvc * (t + v) + b

    z = _pelu(jnp.dot(px, wt1_ref[...],
                      preferred_element_type=jnp.float32) + bt1_ref[...])
    z = _pelu(jnp.dot(z, wt2_ref[...],
                      preferred_element_type=jnp.float32) + bt2_ref[...])
    z = _pelu(cconv(z, w4_ref[...], b4_ref[...]))
    z = _pelu(cconv(z, w5_ref[...], b5_ref[...]))
    o_ref[...] = jnp.dot(z, wf1_ref[...],
                         preferred_element_type=jnp.float32) + bf1_ref[...]


def _coarse(segsum, cnt, sq, mc2, Wt1, bt1, Wt2, bt2, W4, b4, W5, b5,
            Wf1, bf1):
    return pl.pallas_call(
        _coarse_kernel,
        out_shape=jax.ShapeDtypeStruct((SP, 1000), jnp.float32),
    )(segsum, cnt, sq, mc2, Wt1, bt1.reshape(1, 32), Wt2,
      bt2.reshape(1, 128), W4, b4.reshape(1, 128), W5, b5.reshape(1, 128),
      Wf1, bf1.reshape(1, 1000))


def _elu(x):
    return jnp.where(x > 0, x, jnp.expm1(x))


def _mm_kernel(a_ref, w_ref, b_ref, o_ref):
    o_ref[...] = jnp.dot(a_ref[...], w_ref[...],
                         preferred_element_type=jnp.float32) + b_ref[...]


def _mm(a, w, b):
    m, k = a.shape
    n = w.shape[1]
    blk = 1000
    return pl.pallas_call(
        _mm_kernel,
        grid=(m // blk,),
        in_specs=[pl.BlockSpec((blk, k), lambda i: (i, 0)),
                  pl.BlockSpec((k, n), lambda i: (0, 0)),
                  pl.BlockSpec((1, n), lambda i: (0, 0))],
        out_specs=pl.BlockSpec((blk, n), lambda i: (i, 0)),
        out_shape=jax.ShapeDtypeStruct((m, n), jnp.float32),
    )(a, w, b.reshape(1, -1))


def _rowmask(i):
    rows = i * 1024 + lax.broadcasted_iota(jnp.int32, (1024, 1), 0)
    return rows < N


def _dinv_of(d_ref):
    return lax.rsqrt(d_ref[0, :] + d_ref[1, :] + 1.0)[:, None]


def _convin_kernel(a_ref, d_ref, w_ref, o_ref):
    g = _dinv_of(d_ref) * jnp.dot(a_ref[...], w_ref[...],
                                  preferred_element_type=jnp.float32)
    o_ref[...] = jnp.where(_rowmask(pl.program_id(0)), g, 0.0)


def _convin(x, deg2, W):
    return pl.pallas_call(
        _convin_kernel,
        grid=(10,),
        in_specs=[pl.BlockSpec((1024, 128), lambda i: (i, 0)),
                  pl.BlockSpec((2, 1024), lambda i: (0, i)),
                  pl.BlockSpec((128, 128), lambda i: (0, 0))],
        out_specs=pl.BlockSpec((1024, 128), lambda i: (i, 0)),
        out_shape=jax.ShapeDtypeStruct((_ACC_ROWS, 128), jnp.float32),
    )(x, deg2, W)


def _convmid_kernel(sp_ref, g_ref, d_ref, b_ref, w_ref, o_ref):
    dv = _dinv_of(d_ref)
    z = dv * (sp_ref[...] + g_ref[...]) + b_ref[...]
    u = jnp.where(z > 0, z, jnp.exp(z) - 1.0)
    g = dv * jnp.dot(u, w_ref[...], preferred_element_type=jnp.float32)
    o_ref[...] = jnp.where(_rowmask(pl.program_id(0)), g, 0.0)


def _convmid(sp, g, deg2, b, W):
    return pl.pallas_call(
        _convmid_kernel,
        grid=(10,),
        in_specs=[pl.BlockSpec((1024, 128), lambda i: (i, 0)),
                  pl.BlockSpec((1024, 128), lambda i: (i, 0)),
                  pl.BlockSpec((2, 1024), lambda i: (0, i)),
                  pl.BlockSpec((1, 128), lambda i: (0, 0)),
                  pl.BlockSpec((128, 128), lambda i: (0, 0))],
        out_specs=pl.BlockSpec((1024, 128), lambda i: (i, 0)),
        out_shape=jax.ShapeDtypeStruct((_ACC_ROWS, 128), jnp.float32),
    )(sp, g, deg2, b.reshape(1, 128), W)


def kernel(x, adj, num_graphs, in_batch, cluster, W1, b1, W2, b2, W3, b3,
           Wt1, bt1, Wt2, bt2, W4, b4, W5, b5, Wf1, bf1, Wf2, bf2, Wf3, bf3):
    src, dst = adj[0], adj[1]
    src2d = jnp.concatenate(
        [src, jnp.zeros((_EPAD - E,), src.dtype)]).reshape(_EPAD // 128, 128)
    dst2d = jnp.concatenate(
        [dst, jnp.full((_EPAD - E,), N, dst.dtype)]).reshape(_EPAD // 128, 128)
    seg_pad = jnp.concatenate(
        [in_batch * CN + cluster,
         jnp.full((_ACC_ROWS - N,), NG * CN, jnp.int32)])

    deg1, mcnt1 = _stats_pass(src2d, dst2d, seg_pad)
    deg2 = deg1.reshape(_NC, _ACC_ROWS)
    mcnt2 = mcnt1.reshape(_NC, _MWORDS)

    g1 = _convin(x, deg2, W1)
    sp1 = _edge_pass(g1, src2d, dst2d)
    g2 = _convmid(sp1, g1, deg2, b1, W2)
    sp2 = _edge_pass(g2, src2d, dst2d)
    g3 = _convmid(sp2, g2, deg2, b2, W3)
    sp3 = _edge_pass(g3, src2d, dst2d)

    segsum, cnt1, sq1 = _tcpool(sp3[:N], g3[:N], deg2, b3, seg_pad[:N])
    z = _coarse(segsum, cnt1, sq1, mcnt2.reshape(_NC, SP, SP),
                Wt1, bt1, Wt2, bt2, W4, b4, W5, b5, Wf1, bf1)
    k = z[:NG * CN].reshape(-1, CN)
    k = _elu(_mm(k, Wf2, bf2))
    k = _mm(k, Wf3, bf3)
    return k


# speedup vs baseline: 35.8057x; 1.0160x over previous
import functools

import jax
import jax.numpy as jnp
from jax import lax
from jax.experimental import pallas as pl
from jax.experimental.pallas import tpu as pltpu
from jax.experimental.pallas import tpu_sc as plsc

N = 10000
E = 320000
CN = 50
NG = 10
SP = 512

_NC, _NS = 2, 16
_NW = _NC * _NS
_EROWS = 80
_EPT = _EROWS * 128
_EPAD = _NW * _EPT
_ACC_ROWS = 10240
_RPT = _ACC_ROWS // _NS

_sc_mesh = plsc.VectorSubcoreMesh(core_axis_name="c", subcore_axis_name="s",
                                  num_cores=_NC, num_subcores=_NS)


def _zero_rows(buf, rows, cols):
    def zrow(i, _):
        for l in range(cols // 16):
            buf[i, pl.ds(l * 16, 16)] = jnp.zeros((16,), jnp.float32)
        return 0
    lax.fori_loop(0, rows, zrow, 0)


_ER = _EPAD // _NS // 128
_WIN = 2560
_ZROW = N


def _edge_body(g_hbm, src_hbm, dst_hbm, out_hbm,
               src_v, dst_v, rows_a, rows_b, zbuf, acc, sem_a, sem_b):
    c = lax.axis_index("c")
    s = lax.axis_index("s")

    _zero_rows(zbuf, 128, 128)

    def do_phase(base):
        pltpu.sync_copy(zbuf, acc.at[pl.ds(s * 160, 128)])
        pltpu.sync_copy(zbuf.at[pl.ds(0, 32)],
                        acc.at[pl.ds(s * 160 + 128, 32)])
        plsc.subcore_barrier()

        for b in range(2):
            pltpu.sync_copy(src_hbm.at[pl.ds(s * _ER + b * 80, 80)], src_v)
            pltpu.sync_copy(dst_hbm.at[pl.ds(s * _ER + b * 80, 80)], dst_v)

            def remap(j, _):
                for l in range(8):
                    sl = pl.ds(l * 16, 16)
                    dv = dst_v[j, sl]
                    sv = src_v[j, sl]
                    loc = dv - base
                    valid = (loc >= 0) & (loc < _WIN)
                    lane = l * 16 + lax.iota(jnp.int32, 16)
                    dst_v[j, sl] = jnp.where(valid, loc, s * 160 + lane)
                    src_v[j, sl] = jnp.where(valid, sv, _ZROW + lane)
                return 0
            lax.fori_loop(0, 80, remap, 0)

            pltpu.async_copy(g_hbm.at[src_v.at[0]], rows_a, sem_a)

            def pair(jj, _):
                j = jj * 2
                pltpu.make_async_copy(g_hbm.at[src_v.at[j]],
                                      rows_a, sem_a).wait()
                pltpu.async_copy(g_hbm.at[src_v.at[j + 1]], rows_b, sem_b)
                pltpu.sync_copy(rows_a, acc.at[dst_v.at[j]], add=True)
                pltpu.make_async_copy(g_hbm.at[src_v.at[j + 1]],
                                      rows_b, sem_b).wait()

                @pl.when(jj < 39)
                def _():
                    pltpu.async_copy(g_hbm.at[src_v.at[j + 2]], rows_a, sem_a)

                pltpu.sync_copy(rows_b, acc.at[dst_v.at[j + 1]], add=True)
                return 0

            lax.fori_loop(0, 40, pair, 0)

        plsc.subcore_barrier()
        pltpu.sync_copy(acc.at[pl.ds(s * 160, 160)],
                        out_hbm.at[pl.ds(base + s * 160, 160)])
        plsc.subcore_barrier()

    @pl.when(c == 0)
    def _():
        do_phase(0)
        do_phase(_WIN)

    @pl.when(c == 1)
    def _():
        do_phase(2 * _WIN)
        do_phase(3 * _WIN)


_edge_pass = functools.partial(
    pl.kernel,
    out_type=jax.ShapeDtypeStruct((_ACC_ROWS, 128), jnp.float32),
    mesh=_sc_mesh,
    scratch_types=[
        pltpu.VMEM((80, 128), jnp.int32),
        pltpu.VMEM((80, 128), jnp.int32),
        pltpu.VMEM((128, 128), jnp.float32),
        pltpu.VMEM((128, 128), jnp.float32),
        pltpu.VMEM((128, 128), jnp.float32),
        pltpu.VMEM_SHARED((_WIN, 128), jnp.float32),
        pltpu.SemaphoreType.DMA,
        pltpu.SemaphoreType.DMA,
    ],
)(_edge_body)


_MWORDS = SP * SP
_MPT = _MWORDS // _NS


def _stats_body(src_hbm, dst_hbm, seg_hbm, deg_hbm, mcnt_hbm,
                src_v, dst_v, code_v, ps_v, pd_v, ones_v, zv,
                sem_g, sem_h, accdeg, accm):
    c = lax.axis_index("c")
    s = lax.axis_index("s")
    w = c * _NS + s

    def zo(i, _):
        zv[pl.ds(i * 16, 16)] = jnp.zeros((16,), jnp.float32)
        return 0
    lax.fori_loop(0, 128, zo, 0)
    for l in range(8):
        ones_v[pl.ds(l * 16, 16)] = jnp.ones((16,), jnp.float32)

    pltpu.sync_copy(zv.at[pl.ds(0, 640)], accdeg.at[pl.ds(s * 640, 640)])
    for k in range(_MPT // 2048):
        pltpu.sync_copy(zv, accm.at[pl.ds(s * _MPT + k * 2048, 2048)])

    pltpu.sync_copy(src_hbm.at[pl.ds(w * _EROWS, _EROWS)], src_v)
    pltpu.sync_copy(dst_hbm.at[pl.ds(w * _EROWS, _EROWS)], dst_v)
    plsc.subcore_barrier()

    def degj(j, _):
        pltpu.sync_copy(ones_v, accdeg.at[dst_v.at[j]], add=True)
        return 0
    lax.fori_loop(0, _EROWS, degj, 0)

    def codej(j, _):
        pltpu.async_copy(seg_hbm.at[src_v.at[j]], ps_v, sem_g)
        pltpu.async_copy(seg_hbm.at[dst_v.at[j]], pd_v, sem_h)
        pltpu.make_async_copy(seg_hbm.at[src_v.at[j]], ps_v, sem_g).wait()
        pltpu.make_async_copy(seg_hbm.at[dst_v.at[j]], pd_v, sem_h).wait()
        for l in range(8):
            ps = ps_v[pl.ds(l * 16, 16)]
            pd = pd_v[pl.ds(l * 16, 16)]
            code_v[j, pl.ds(l * 16, 16)] = ps * SP + pd
        return 0
    lax.fori_loop(0, _EROWS, codej, 0)

    def mj(j, _):
        pltpu.sync_copy(ones_v, accm.at[code_v.at[j]], add=True)
        return 0
    lax.fori_loop(0, _EROWS, mj, 0)

    plsc.subcore_barrier()
    pltpu.sync_copy(accdeg.at[pl.ds(s * 640, 640)],
                    deg_hbm.at[pl.ds(c * _ACC_ROWS + s * 640, 640)])
    pltpu.sync_copy(accm.at[pl.ds(s * _MPT, _MPT)],
                    mcnt_hbm.at[pl.ds(c * _MWORDS + s * _MPT, _MPT)])


_stats_pass = functools.partial(
    pl.kernel,
    out_type=(jax.ShapeDtypeStruct((_NC * _ACC_ROWS,), jnp.float32),
              jax.ShapeDtypeStruct((_NC * _MWORDS,), jnp.float32)),
    mesh=_sc_mesh,
    scratch_types=[
        pltpu.VMEM((_EROWS, 128), jnp.int32),
        pltpu.VMEM((_EROWS, 128), jnp.int32),
        pltpu.VMEM((_EROWS, 128), jnp.int32),
        pltpu.VMEM((128,), jnp.int32),
        pltpu.VMEM((128,), jnp.int32),
        pltpu.VMEM((128,), jnp.float32),
        pltpu.VMEM((2048,), jnp.float32),
        pltpu.SemaphoreType.DMA,
        pltpu.SemaphoreType.DMA,
        pltpu.VMEM_SHARED((_ACC_ROWS,), jnp.float32),
        pltpu.VMEM_SHARED((_MWORDS,), jnp.float32),
    ],
)(_stats_body)


def _tcpool_kernel(sp_ref, g_ref, d_ref, b_ref, seg_ref,
                   sum_ref, cnt_ref, sq_ref):
    i = pl.program_id(0)
    h = lax.rsqrt(d_ref[0, 0, :] + d_ref[0, 1, :] + 1.0)[:, None] \
        * (sp_ref[...] + g_ref[...]) + b_ref[...]
    oh = jnp.where(seg_ref[0, 0, :][:, None] ==
                   lax.broadcasted_iota(jnp.int32, (1000, SP), 1), 1.0, 0.0)
    psum = lax.dot_general(oh, h, (((0,), (0,)), ((), ())),
                           preferred_element_type=jnp.float32)
    pcnt = jnp.sum(oh, axis=0, keepdims=True)

    @pl.when(i == 0)
    def _():
        sum_ref[...] = jnp.zeros_like(sum_ref)
        cnt_ref[...] = jnp.zeros_like(cnt_ref)
        sq_ref[...] = jnp.zeros_like(sq_ref)

    sum_ref[...] += psum
    cnt_ref[...] += pcnt
    sq_ref[...] += jnp.sum(h * h, axis=0, keepdims=True)


def _tcpool(sp, g, deg2, b, seg):
    return pl.pallas_call(
        _tcpool_kernel,
        grid=(N // 1000,),
        in_specs=[pl.BlockSpec((1000, 128), lambda i: (i, 0)),
                  pl.BlockSpec((1000, 128), lambda i: (i, 0)),
                  pl.BlockSpec((1, 2, 1000), lambda i: (i, 0, 0)),
                  pl.BlockSpec((1, 128), lambda i: (0, 0)),
                  pl.BlockSpec((1, 1, 1000), lambda i: (i, 0, 0))],
        out_specs=[pl.BlockSpec((SP, 128), lambda i: (0, 0)),
                   pl.BlockSpec((1, SP), lambda i: (0, 0)),
                   pl.BlockSpec((1, 128), lambda i: (0, 0))],
        out_shape=[jax.ShapeDtypeStruct((SP, 128), jnp.float32),
                   jax.ShapeDtypeStruct((1, SP), jnp.float32),
                   jax.ShapeDtypeStruct((1, 128), jnp.float32)],
    )(sp, g, deg2[:, :N].reshape(2, N // 1000, 1000).transpose(1, 0, 2),
      b.reshape(1, 128), seg.reshape(N // 1000, 1, 1000))


def _pelu(x):
    return jnp.where(x > 0, x, jnp.exp(x) - 1.0)


def _coarse_kernel(segsum_ref, cnt_ref, sq_ref, mc_ref,
                   wt1_ref, bt1_ref, wt2_ref, bt2_ref,
                   w4_ref, b4_ref, w5_ref, b5_ref, wf1_ref, bf1_ref, o_ref):
    mean = jnp.sum(segsum_ref[...], axis=0, keepdims=True) / N
    var = sq_ref[...] / N - mean * mean
    rstd = lax.rsqrt(var + 1e-5)
    cnt = cnt_ref[0, :][:, None]
    px = (segsum_ref[...] - cnt * mean) / jnp.maximum(cnt, 1.0) * rstd

    mc = mc_ref[0] + mc_ref[1]
    rid = lax.broadcasted_iota(jnp.int32, (SP, SP), 0)
    cid = lax.broadcasted_iota(jnp.int32, (SP, SP), 1)
    keep = (mc > 0) & (rid != cid) & (rid < NG * CN) & (cid < NG * CN)
    M = jnp.where(keep, 1.0, 0.0)
    degc = jnp.sum(M, axis=0, keepdims=True) + 1.0
    dinvc = lax.rsqrt(degc).reshape(SP, 1)

    def cconv(u, W, b):
        v = dinvc * jnp.dot(u, W, preferred_element_type=jnp.float32)
        t = lax.dot_general(M, v, (((0,), (0,)), ((), ())),
                            preferred_element_type=jnp.float32)
        return dinvc * (t + v) + b

    z = _pelu(jnp.dot(px, wt1_ref[...],
                      preferred_element_type=jnp.float32) + bt1_ref[...])
    z = _pelu(jnp.dot(z, wt2_ref[...],
                      preferred_element_type=jnp.float32) + bt2_ref[...])
    z = _pelu(cconv(z, w4_ref[...], b4_ref[...]))
    z = _pelu(cconv(z, w5_ref[...], b5_ref[...]))
    o_ref[...] = jnp.dot(z, wf1_ref[...],
                         preferred_element_type=jnp.float32) + bf1_ref[...]


def _coarse(segsum, cnt, sq, mc2, Wt1, bt1, Wt2, bt2, W4, b4, W5, b5,
            Wf1, bf1):
    return pl.pallas_call(
        _coarse_kernel,
        out_shape=jax.ShapeDtypeStruct((SP, 1000), jnp.float32),
    )(segsum, cnt, sq, mc2, Wt1, bt1.reshape(1, 32), Wt2,
      bt2.reshape(1, 128), W4, b4.reshape(1, 128), W5, b5.reshape(1, 128),
      Wf1, bf1.reshape(1, 1000))


def _elu(x):
    return jnp.where(x > 0, x, jnp.expm1(x))


def _mm_kernel(a_ref, w_ref, b_ref, o_ref):
    o_ref[...] = jnp.dot(a_ref[...], w_ref[...],
                         preferred_element_type=jnp.float32) + b_ref[...]


def _mm(a, w, b):
    m, k = a.shape
    n = w.shape[1]
    blk = 1000
    return pl.pallas_call(
        _mm_kernel,
        grid=(m // blk,),
        in_specs=[pl.BlockSpec((blk, k), lambda i: (i, 0)),
                  pl.BlockSpec((k, n), lambda i: (0, 0)),
                  pl.BlockSpec((1, n), lambda i: (0, 0))],
        out_specs=pl.BlockSpec((blk, n), lambda i: (i, 0)),
        out_shape=jax.ShapeDtypeStruct((m, n), jnp.float32),
    )(a, w, b.reshape(1, -1))


def _rowmask(i):
    rows = i * 1024 + lax.broadcasted_iota(jnp.int32, (1024, 1), 0)
    return rows < N


def _dinv_of(d_ref):
    return lax.rsqrt(d_ref[0, :] + d_ref[1, :] + 1.0)[:, None]


def _convin_kernel(a_ref, d_ref, w_ref, o_ref):
    g = _dinv_of(d_ref) * jnp.dot(a_ref[...], w_ref[...],
                                  preferred_element_type=jnp.float32)
    o_ref[...] = jnp.where(_rowmask(pl.program_id(0)), g, 0.0)


def _convin(x, deg2, W):
    return pl.pallas_call(
        _convin_kernel,
        grid=(10,),
        in_specs=[pl.BlockSpec((1024, 128), lambda i: (i, 0)),
                  pl.BlockSpec((2, 1024), lambda i: (0, i)),
                  pl.BlockSpec((128, 128), lambda i: (0, 0))],
        out_specs=pl.BlockSpec((1024, 128), lambda i: (i, 0)),
        out_shape=jax.ShapeDtypeStruct((_ACC_ROWS, 128), jnp.float32),
    )(x, deg2, W)


def _convmid_kernel(sp_ref, g_ref, d_ref, b_ref, w_ref, o_ref):
    dv = _dinv_of(d_ref)
    z = dv * (sp_ref[...] + g_ref[...]) + b_ref[...]
    u = jnp.where(z > 0, z, jnp.exp(z) - 1.0)
    g = dv * jnp.dot(u, w_ref[...], preferred_element_type=jnp.float32)
    o_ref[...] = jnp.where(_rowmask(pl.program_id(0)), g, 0.0)


def _convmid(sp, g, deg2, b, W):
    return pl.pallas_call(
        _convmid_kernel,
        grid=(10,),
        in_specs=[pl.BlockSpec((1024, 128), lambda i: (i, 0)),
                  pl.BlockSpec((1024, 128), lambda i: (i, 0)),
                  pl.BlockSpec((2, 1024), lambda i: (0, i)),
                  pl.BlockSpec((1, 128), lambda i: (0, 0)),
                  pl.BlockSpec((128, 128), lambda i: (0, 0))],
        out_specs=pl.BlockSpec((1024, 128), lambda i: (i, 0)),
        out_shape=jax.ShapeDtypeStruct((_ACC_ROWS, 128), jnp.float32),
    )(sp, g, deg2, b.reshape(1, 128), W)


def kernel(x, adj, num_graphs, in_batch, cluster, W1, b1, W2, b2, W3, b3,
           Wt1, bt1, Wt2, bt2, W4, b4, W5, b5, Wf1, bf1, Wf2, bf2, Wf3, bf3):
    src, dst = adj[0], adj[1]
    src2d = jnp.concatenate(
        [src, jnp.zeros((_EPAD - E,), src.dtype)]).reshape(_EPAD // 128, 128)
    dst2d = jnp.concatenate(
        [dst, jnp.full((_EPAD - E,), N, dst.dtype)]).reshape(_EPAD // 128, 128)
    seg_pad = jnp.concatenate(
        [in_batch * CN + cluster,
         jnp.full((_ACC_ROWS - N,), NG * CN, jnp.int32)])

    deg1, mcnt1 = _stats_pass(src2d, dst2d, seg_pad)
    deg2 = deg1.reshape(_NC, _ACC_ROWS)
    mcnt2 = mcnt1.reshape(_NC, _MWORDS)

    g1 = _convin(x, deg2, W1)
    sp1 = _edge_pass(g1, src2d, dst2d)
    g2 = _convmid(sp1, g1, deg2, b1, W2)
    sp2 = _edge_pass(g2, src2d, dst2d)
    g3 = _convmid(sp2, g2, deg2, b2, W3)
    sp3 = _edge_pass(g3, src2d, dst2d)

    segsum, cnt1, sq1 = _tcpool(sp3[:N], g3[:N], deg2, b3, seg_pad[:N])
    z = _coarse(segsum, cnt1, sq1, mcnt2.reshape(_NC, SP, SP),
                Wt1, bt1, Wt2, bt2, W4, b4, W5, b5, Wf1, bf1)
    k = z[:NG * CN].reshape(-1, CN)
    k = _elu(_mm(k, Wf2, bf2))
    k = _mm(k, Wf3, bf3)
    return k


# speedup vs baseline: 35.9045x; 1.0028x over previous
import functools

import jax
import jax.numpy as jnp
from jax import lax
from jax.experimental import pallas as pl
from jax.experimental.pallas import tpu as pltpu
from jax.experimental.pallas import tpu_sc as plsc

N = 10000
E = 320000
CN = 50
NG = 10
SP = 512

_NC, _NS = 2, 16
_NW = _NC * _NS
_EROWS = 80
_EPT = _EROWS * 128
_EPAD = _NW * _EPT
_ACC_ROWS = 10240
_RPT = _ACC_ROWS // _NS

_sc_mesh = plsc.VectorSubcoreMesh(core_axis_name="c", subcore_axis_name="s",
                                  num_cores=_NC, num_subcores=_NS)


def _zero_rows(buf, rows, cols):
    def zrow(i, _):
        for l in range(cols // 16):
            buf[i, pl.ds(l * 16, 16)] = jnp.zeros((16,), jnp.float32)
        return 0
    lax.fori_loop(0, rows, zrow, 0)


_ER = _EPAD // _NS // 128
_WIN = 2560
_ZROW = N


def _edge_body(g_hbm, src_hbm, dst_hbm, out_hbm,
               src_v, dst_v, rows_a, rows_b, zbuf, acc, sem_a, sem_b):
    c = lax.axis_index("c")
    s = lax.axis_index("s")

    _zero_rows(zbuf, 128, 128)

    def do_phase(base):
        pltpu.sync_copy(zbuf, acc.at[pl.ds(s * 160, 128)])
        pltpu.sync_copy(zbuf.at[pl.ds(0, 32)],
                        acc.at[pl.ds(s * 160 + 128, 32)])
        plsc.subcore_barrier()

        for b in range(2):
            pltpu.sync_copy(src_hbm.at[pl.ds(s * _ER + b * 80, 80)], src_v)
            pltpu.sync_copy(dst_hbm.at[pl.ds(s * _ER + b * 80, 80)], dst_v)

            def remap(j, _):
                for l in range(8):
                    sl = pl.ds(l * 16, 16)
                    dv = dst_v[j, sl]
                    sv = src_v[j, sl]
                    loc = dv - base
                    valid = (loc >= 0) & (loc < _WIN)
                    lane = l * 16 + lax.iota(jnp.int32, 16)
                    dst_v[j, sl] = jnp.where(valid, loc, s * 160 + lane)
                    src_v[j, sl] = jnp.where(valid, sv, _ZROW + lane)
                return 0
            lax.fori_loop(0, 80, remap, 0)

            pltpu.async_copy(g_hbm.at[src_v.at[0]], rows_a, sem_a)

            def pair(jj, _):
                j = jj * 2
                pltpu.make_async_copy(g_hbm.at[src_v.at[j]],
                                      rows_a, sem_a).wait()
                pltpu.async_copy(g_hbm.at[src_v.at[j + 1]], rows_b, sem_b)
                pltpu.sync_copy(rows_a, acc.at[dst_v.at[j]], add=True)
                pltpu.make_async_copy(g_hbm.at[src_v.at[j + 1]],
                                      rows_b, sem_b).wait()

                @pl.when(jj < 39)
                def _():
                    pltpu.async_copy(g_hbm.at[src_v.at[j + 2]], rows_a, sem_a)

                pltpu.sync_copy(rows_b, acc.at[dst_v.at[j + 1]], add=True)
                return 0

            lax.fori_loop(0, 40, pair, 0)

        plsc.subcore_barrier()
        pltpu.sync_copy(acc.at[pl.ds(s * 160, 160)],
                        out_hbm.at[pl.ds(base + s * 160, 160)])
        plsc.subcore_barrier()

    @pl.when(c == 0)
    def _():
        do_phase(0)
        do_phase(_WIN)

    @pl.when(c == 1)
    def _():
        do_phase(2 * _WIN)
        do_phase(3 * _WIN)


_edge_pass = functools.partial(
    pl.kernel,
    out_type=jax.ShapeDtypeStruct((_ACC_ROWS, 128), jnp.float32),
    mesh=_sc_mesh,
    scratch_types=[
        pltpu.VMEM((80, 128), jnp.int32),
        pltpu.VMEM((80, 128), jnp.int32),
        pltpu.VMEM((128, 128), jnp.float32),
        pltpu.VMEM((128, 128), jnp.float32),
        pltpu.VMEM((128, 128), jnp.float32),
        pltpu.VMEM_SHARED((_WIN, 128), jnp.float32),
        pltpu.SemaphoreType.DMA,
        pltpu.SemaphoreType.DMA,
    ],
)(_edge_body)


_MWORDS = SP * SP
_MPT = _MWORDS // _NS


def _stats_body(src_hbm, dst_hbm, seg_hbm, deg_hbm, mcnt_hbm,
                src_v, dst_v, code_v, ps_v, pd_v, ones_v, zv,
                sem_g, sem_h, sem_s, accdeg, accm):
    c = lax.axis_index("c")
    s = lax.axis_index("s")
    w = c * _NS + s

    def zo(i, _):
        zv[pl.ds(i * 16, 16)] = jnp.zeros((16,), jnp.float32)
        return 0
    lax.fori_loop(0, 128, zo, 0)
    for l in range(8):
        ones_v[pl.ds(l * 16, 16)] = jnp.ones((16,), jnp.float32)

    pltpu.sync_copy(zv.at[pl.ds(0, 640)], accdeg.at[pl.ds(s * 640, 640)])
    for k in range(_MPT // 2048):
        pltpu.sync_copy(zv, accm.at[pl.ds(s * _MPT + k * 2048, 2048)])

    pltpu.sync_copy(src_hbm.at[pl.ds(w * _EROWS, _EROWS)], src_v)
    pltpu.sync_copy(dst_hbm.at[pl.ds(w * _EROWS, _EROWS)], dst_v)
    plsc.subcore_barrier()

    def degj(j, _):
        pltpu.async_copy(ones_v, accdeg.at[dst_v.at[j]], sem_s, add=True)
        return 0
    lax.fori_loop(0, _EROWS, degj, 0)

    def degw(j, _):
        pltpu.make_async_copy(ones_v, accdeg.at[dst_v.at[j]], sem_s).wait()
        return 0
    lax.fori_loop(0, _EROWS, degw, 0)

    def codej(j, _):
        pltpu.async_copy(seg_hbm.at[src_v.at[j]], ps_v, sem_g)
        pltpu.async_copy(seg_hbm.at[dst_v.at[j]], pd_v, sem_h)
        pltpu.make_async_copy(seg_hbm.at[src_v.at[j]], ps_v, sem_g).wait()
        pltpu.make_async_copy(seg_hbm.at[dst_v.at[j]], pd_v, sem_h).wait()
        for l in range(8):
            ps = ps_v[pl.ds(l * 16, 16)]
            pd = pd_v[pl.ds(l * 16, 16)]
            code_v[j, pl.ds(l * 16, 16)] = ps * SP + pd
        return 0
    lax.fori_loop(0, _EROWS, codej, 0)

    def mj(j, _):
        pltpu.async_copy(ones_v, accm.at[code_v.at[j]], sem_s, add=True)
        return 0
    lax.fori_loop(0, _EROWS, mj, 0)

    def mw(j, _):
        pltpu.make_async_copy(ones_v, accm.at[code_v.at[j]], sem_s).wait()
        return 0
    lax.fori_loop(0, _EROWS, mw, 0)

    plsc.subcore_barrier()
    pltpu.sync_copy(accdeg.at[pl.ds(s * 640, 640)],
                    deg_hbm.at[pl.ds(c * _ACC_ROWS + s * 640, 640)])
    pltpu.sync_copy(accm.at[pl.ds(s * _MPT, _MPT)],
                    mcnt_hbm.at[pl.ds(c * _MWORDS + s * _MPT, _MPT)])


_stats_pass = functools.partial(
    pl.kernel,
    out_type=(jax.ShapeDtypeStruct((_NC * _ACC_ROWS,), jnp.float32),
              jax.ShapeDtypeStruct((_NC * _MWORDS,), jnp.float32)),
    mesh=_sc_mesh,
    scratch_types=[
        pltpu.VMEM((_EROWS, 128), jnp.int32),
        pltpu.VMEM((_EROWS, 128), jnp.int32),
        pltpu.VMEM((_EROWS, 128), jnp.int32),
        pltpu.VMEM((128,), jnp.int32),
        pltpu.VMEM((128,), jnp.int32),
        pltpu.VMEM((128,), jnp.float32),
        pltpu.VMEM((2048,), jnp.float32),
        pltpu.SemaphoreType.DMA,
        pltpu.SemaphoreType.DMA,
        pltpu.SemaphoreType.DMA,
        pltpu.VMEM_SHARED((_ACC_ROWS,), jnp.float32),
        pltpu.VMEM_SHARED((_MWORDS,), jnp.float32),
    ],
)(_stats_body)


def _tcpool_kernel(sp_ref, g_ref, d_ref, b_ref, seg_ref,
                   sum_ref, cnt_ref, sq_ref):
    i = pl.program_id(0)
    h = lax.rsqrt(d_ref[0, 0, :] + d_ref[0, 1, :] + 1.0)[:, None] \
        * (sp_ref[...] + g_ref[...]) + b_ref[...]
    oh = jnp.where(seg_ref[0, 0, :][:, None] ==
                   lax.broadcasted_iota(jnp.int32, (1000, SP), 1), 1.0, 0.0)
    psum = lax.dot_general(oh, h, (((0,), (0,)), ((), ())),
                           preferred_element_type=jnp.float32)
    pcnt = jnp.sum(oh, axis=0, keepdims=True)

    @pl.when(i == 0)
    def _():
        sum_ref[...] = jnp.zeros_like(sum_ref)
        cnt_ref[...] = jnp.zeros_like(cnt_ref)
        sq_ref[...] = jnp.zeros_like(sq_ref)

    sum_ref[...] += psum
    cnt_ref[...] += pcnt
    sq_ref[...] += jnp.sum(h * h, axis=0, keepdims=True)


def _tcpool(sp, g, deg2, b, seg):
    return pl.pallas_call(
        _tcpool_kernel,
        grid=(N // 1000,),
        in_specs=[pl.BlockSpec((1000, 128), lambda i: (i, 0)),
                  pl.BlockSpec((1000, 128), lambda i: (i, 0)),
                  pl.BlockSpec((1, 2, 1000), lambda i: (i, 0, 0)),
                  pl.BlockSpec((1, 128), lambda i: (0, 0)),
                  pl.BlockSpec((1, 1, 1000), lambda i: (i, 0, 0))],
        out_specs=[pl.BlockSpec((SP, 128), lambda i: (0, 0)),
                   pl.BlockSpec((1, SP), lambda i: (0, 0)),
                   pl.BlockSpec((1, 128), lambda i: (0, 0))],
        out_shape=[jax.ShapeDtypeStruct((SP, 128), jnp.float32),
                   jax.ShapeDtypeStruct((1, SP), jnp.float32),
                   jax.ShapeDtypeStruct((1, 128), jnp.float32)],
    )(sp, g, deg2[:, :N].reshape(2, N // 1000, 1000).transpose(1, 0, 2),
      b.reshape(1, 128), seg.reshape(N // 1000, 1, 1000))


def _pelu(x):
    return jnp.where(x > 0, x, jnp.exp(x) - 1.0)


def _coarse_kernel(segsum_ref, cnt_ref, sq_ref, mc_ref,
                   wt1_ref, bt1_ref, wt2_ref, bt2_ref,
                   w4_ref, b4_ref, w5_ref, b5_ref, wf1_ref, bf1_ref, o_ref):
    mean = jnp.sum(segsum_ref[...], axis=0, keepdims=True) / N
    var = sq_ref[...] / N - mean * mean
    rstd = lax.rsqrt(var + 1e-5)
    cnt = cnt_ref[0, :][:, None]
    px = (segsum_ref[...] - cnt * mean) / jnp.maximum(cnt, 1.0) * rstd

    mc = mc_ref[0] + mc_ref[1]
    rid = lax.broadcasted_iota(jnp.int32, (SP, SP), 0)
    cid = lax.broadcasted_iota(jnp.int32, (SP, SP), 1)
    keep = (mc > 0) & (rid != cid) & (rid < NG * CN) & (cid < NG * CN)
    M = jnp.where(keep, 1.0, 0.0)
    degc = jnp.sum(M, axis=0, keepdims=True) + 1.0
    dinvc = lax.rsqrt(degc).reshape(SP, 1)

    def cconv(u, W, b):
        v = dinvc * jnp.dot(u, W, preferred_element_type=jnp.float32)
        t = lax.dot_general(M, v, (((0,), (0,)), ((), ())),
                            preferred_element_type=jnp.float32)
        return dinvc * (t + v) + b

    z = _pelu(jnp.dot(px, wt1_ref[...],
                      preferred_element_type=jnp.float32) + bt1_ref[...])
    z = _pelu(jnp.dot(z, wt2_ref[...],
                      preferred_element_type=jnp.float32) + bt2_ref[...])
    z = _pelu(cconv(z, w4_ref[...], b4_ref[...]))
    z = _pelu(cconv(z, w5_ref[...], b5_ref[...]))
    o_ref[...] = jnp.dot(z, wf1_ref[...],
                         preferred_element_type=jnp.float32) + bf1_ref[...]


def _coarse(segsum, cnt, sq, mc2, Wt1, bt1, Wt2, bt2, W4, b4, W5, b5,
            Wf1, bf1):
    return pl.pallas_call(
        _coarse_kernel,
        out_shape=jax.ShapeDtypeStruct((SP, 1000), jnp.float32),
    )(segsum, cnt, sq, mc2, Wt1, bt1.reshape(1, 32), Wt2,
      bt2.reshape(1, 128), W4, b4.reshape(1, 128), W5, b5.reshape(1, 128),
      Wf1, bf1.reshape(1, 1000))


def _elu(x):
    return jnp.where(x > 0, x, jnp.expm1(x))


def _mm_kernel(a_ref, w_ref, b_ref, o_ref):
    o_ref[...] = jnp.dot(a_ref[...], w_ref[...],
                         preferred_element_type=jnp.float32) + b_ref[...]


def _mm(a, w, b):
    m, k = a.shape
    n = w.shape[1]
    blk = 1000
    return pl.pallas_call(
        _mm_kernel,
        grid=(m // blk,),
        in_specs=[pl.BlockSpec((blk, k), lambda i: (i, 0)),
                  pl.BlockSpec((k, n), lambda i: (0, 0)),
                  pl.BlockSpec((1, n), lambda i: (0, 0))],
        out_specs=pl.BlockSpec((blk, n), lambda i: (i, 0)),
        out_shape=jax.ShapeDtypeStruct((m, n), jnp.float32),
    )(a, w, b.reshape(1, -1))


def _rowmask(i):
    rows = i * 1024 + lax.broadcasted_iota(jnp.int32, (1024, 1), 0)
    return rows < N


def _dinv_of(d_ref):
    return lax.rsqrt(d_ref[0, :] + d_ref[1, :] + 1.0)[:, None]


def _convin_kernel(a_ref, d_ref, w_ref, o_ref):
    g = _dinv_of(d_ref) * jnp.dot(a_ref[...], w_ref[...],
                                  preferred_element_type=jnp.float32)
    o_ref[...] = jnp.where(_rowmask(pl.program_id(0)), g, 0.0)


def _convin(x, deg2, W):
    return pl.pallas_call(
        _convin_kernel,
        grid=(10,),
        in_specs=[pl.BlockSpec((1024, 128), lambda i: (i, 0)),
                  pl.BlockSpec((2, 1024), lambda i: (0, i)),
                  pl.BlockSpec((128, 128), lambda i: (0, 0))],
        out_specs=pl.BlockSpec((1024, 128), lambda i: (i, 0)),
        out_shape=jax.ShapeDtypeStruct((_ACC_ROWS, 128), jnp.float32),
    )(x, deg2, W)


def _convmid_kernel(sp_ref, g_ref, d_ref, b_ref, w_ref, o_ref):
    dv = _dinv_of(d_ref)
    z = dv * (sp_ref[...] + g_ref[...]) + b_ref[...]
    u = jnp.where(z > 0, z, jnp.exp(z) - 1.0)
    g = dv * jnp.dot(u, w_ref[...], preferred_element_type=jnp.float32)
    o_ref[...] = jnp.where(_rowmask(pl.program_id(0)), g, 0.0)


def _convmid(sp, g, deg2, b, W):
    return pl.pallas_call(
        _convmid_kernel,
        grid=(10,),
        in_specs=[pl.BlockSpec((1024, 128), lambda i: (i, 0)),
                  pl.BlockSpec((1024, 128), lambda i: (i, 0)),
                  pl.BlockSpec((2, 1024), lambda i: (0, i)),
                  pl.BlockSpec((1, 128), lambda i: (0, 0)),
                  pl.BlockSpec((128, 128), lambda i: (0, 0))],
        out_specs=pl.BlockSpec((1024, 128), lambda i: (i, 0)),
        out_shape=jax.ShapeDtypeStruct((_ACC_ROWS, 128), jnp.float32),
    )(sp, g, deg2, b.reshape(1, 128), W)


def kernel(x, adj, num_graphs, in_batch, cluster, W1, b1, W2, b2, W3, b3,
           Wt1, bt1, Wt2, bt2, W4, b4, W5, b5, Wf1, bf1, Wf2, bf2, Wf3, bf3):
    src, dst = adj[0], adj[1]
    src2d = jnp.concatenate(
        [src, jnp.zeros((_EPAD - E,), src.dtype)]).reshape(_EPAD // 128, 128)
    dst2d = jnp.concatenate(
        [dst, jnp.full((_EPAD - E,), N, dst.dtype)]).reshape(_EPAD // 128, 128)
    seg_pad = jnp.concatenate(
        [in_batch * CN + cluster,
         jnp.full((_ACC_ROWS - N,), NG * CN, jnp.int32)])

    deg1, mcnt1 = _stats_pass(src2d, dst2d, seg_pad)
    deg2 = deg1.reshape(_NC, _ACC_ROWS)
    mcnt2 = mcnt1.reshape(_NC, _MWORDS)

    g1 = _convin(x, deg2, W1)
    sp1 = _edge_pass(g1, src2d, dst2d)
    g2 = _convmid(sp1, g1, deg2, b1, W2)
    sp2 = _edge_pass(g2, src2d, dst2d)
    g3 = _convmid(sp2, g2, deg2, b2, W3)
    sp3 = _edge_pass(g3, src2d, dst2d)

    segsum, cnt1, sq1 = _tcpool(sp3[:N], g3[:N], deg2, b3, seg_pad[:N])
    z = _coarse(segsum, cnt1, sq1, mcnt2.reshape(_NC, SP, SP),
                Wt1, bt1, Wt2, bt2, W4, b4, W5, b5, Wf1, bf1)
    k = z[:NG * CN].reshape(-1, CN)
    k = _elu(_mm(k, Wf2, bf2))
    k = _mm(k, Wf3, bf3)
    return k
